# baseline (speedup 1.0000x reference)
import functools

import jax
import jax.numpy as jnp
from jax.experimental import pallas as pl
from jax.experimental.pallas import tpu as pltpu
from jax.experimental.pallas import tpu_sc as plsc

N = 8192
K = 20
NUM_GRAPHS = 8
OUT_DIM = 40

_BIG = 1e30
_BIG2 = 2e30


_WINS = (1536, 2560, N)


def _extract_topk(d, lane, idx_ref):
    cols = []
    for _ in range(K):
        m = jnp.min(d, axis=1, keepdims=True)
        am = jnp.min(jnp.where(d <= m, lane, N), axis=1, keepdims=True)
        cols.append(am)
        d = jnp.where(lane == am, _BIG2, d)
    idx_ref[...] = jnp.concatenate(cols, axis=1)


def _knn_body(info_ref, x_ref, xt_ref, bc_ref, br_ref, idx_ref, *, rows):
    pid = pl.program_id(0)
    start = pl.multiple_of(info_ref[0, pid], 256)
    sel = info_ref[1, pid]
    x = x_ref[...]
    sqc = jnp.sum(x * x, axis=1, keepdims=True)
    bc = bc_ref[...]

    def _tier(win, s):
        xt = xt_ref[:, pl.ds(s, win)] if win < N else xt_ref[...]
        br = br_ref[:, pl.ds(s, win)] if win < N else br_ref[...]
        sqr = jnp.sum(xt * xt, axis=0, keepdims=True)
        d = sqc + sqr - 2.0 * jnp.dot(x, xt, preferred_element_type=jnp.float32)
        d = jnp.where(bc != br, _BIG, d)
        lane = jax.lax.broadcasted_iota(jnp.int32, (rows, win), 1) + s
        _extract_topk(d, lane, idx_ref)

    for t, w in enumerate(_WINS):
        s = start if w < N else 0

        @pl.when(sel == t)
        def _(w=w, s=s):
            _tier(w, s)


def _knn(info, x, xt, batch_col, batch_row, rows=256):
    f = x.shape[1]
    grid = N // rows
    return pl.pallas_call(
        functools.partial(_knn_body, rows=rows),
        grid_spec=pltpu.PrefetchScalarGridSpec(
            num_scalar_prefetch=1,
            grid=(grid,),
            in_specs=[
                pl.BlockSpec((rows, f), lambda i, inf: (i, 0)),
                pl.BlockSpec((f, N), lambda i, inf: (0, 0)),
                pl.BlockSpec((rows, 1), lambda i, inf: (i, 0)),
                pl.BlockSpec((1, N), lambda i, inf: (0, 0)),
            ],
            out_specs=pl.BlockSpec((rows, K), lambda i, inf: (i, 0)),
        ),
        out_shape=jax.ShapeDtypeStruct((N, K), jnp.int32),
    )(info, x, xt, batch_col, batch_row)


def _knn_window_info(batch, rows=256):
    br = batch.reshape(N // rows, rows)
    seg_start = jnp.searchsorted(batch, jnp.arange(NUM_GRAPHS), side="left")
    seg_end = jnp.searchsorted(batch, jnp.arange(NUM_GRAPHS) + 1, side="left")
    cs = seg_start[br[:, 0]]
    ce = seg_end[br[:, -1]]
    s = (cs // 256) * 256
    sel = jnp.full(s.shape, len(_WINS) - 1, jnp.int32)
    for t in range(len(_WINS) - 2, -1, -1):
        st = jnp.minimum(s, N - _WINS[t])
        sel = jnp.where(ce - st <= _WINS[t], t, sel)
    s_final = jnp.minimum(s, N - jnp.asarray(_WINS)[sel])
    return jnp.stack([s_final.astype(jnp.int32), sel])


def _make_sc_gather(b_total, d, chunk):
    info = plsc.get_sparse_core_info()
    nc, ns = info.num_cores, info.num_subcores
    nw = nc * ns
    b_per_w = b_total // nw
    nchunks = b_per_w // chunk
    assert b_per_w % chunk == 0 and chunk % 8 == 0
    mesh = plsc.VectorSubcoreMesh(core_axis_name="c", subcore_axis_name="s")

    @functools.partial(
        pl.kernel,
        mesh=mesh,
        compiler_params=pltpu.CompilerParams(use_tc_tiling_on_sc=False),
        out_type=jax.ShapeDtypeStruct((b_total, d), jnp.float32),
        scratch_types=[
            pltpu.VMEM((chunk,), jnp.int32),
            pltpu.VMEM((chunk, d), jnp.float32),
            pltpu.SemaphoreType.DMA,
        ],
    )
    def gather(table_hbm, idx_hbm, out_hbm, idx_v, rows_v, sem):
        wid = jax.lax.axis_index("s") * nc + jax.lax.axis_index("c")
        base = wid * b_per_w
        for c in range(nchunks):
            off = base + c * chunk
            pltpu.sync_copy(idx_hbm.at[pl.ds(off, chunk)], idx_v)
            pltpu.async_copy(table_hbm.at[idx_v], rows_v, sem).wait()
            pltpu.sync_copy(rows_v, out_hbm.at[pl.ds(off, chunk)])

    return gather


def _pre1(pos_blk, posg_blk, wa_ref, wb_ref, b_ref):
    return (jnp.dot(pos_blk, wa_ref[...], preferred_element_type=jnp.float32)
            + jnp.dot(posg_blk - pos_blk, wb_ref[...], preferred_element_type=jnp.float32)
            + b_ref[...])


def _stats1_body(pg_ref, p_ref, wa_ref, wb_ref, b_ref, s_ref, q_ref):
    first = (pl.program_id(0) == 0) & (pl.program_id(1) == 0)

    @pl.when(first)
    def _():
        s_ref[...] = jnp.zeros_like(s_ref)
        q_ref[...] = jnp.zeros_like(q_ref)

    pre = _pre1(p_ref[...], pg_ref[0], wa_ref, wb_ref, b_ref)
    s_ref[...] += jnp.sum(pre, axis=0, keepdims=True)
    q_ref[...] += jnp.sum(pre * pre, axis=0, keepdims=True)


def _stats1(pos_g, pos_pad, wa, wb, b1, rows=1024):
    fp = pos_pad.shape[1]
    grid = (K, N // rows)
    return pl.pallas_call(
        _stats1_body,
        grid=grid,
        in_specs=[
            pl.BlockSpec((1, rows, fp), lambda k, j: (k, j, 0)),
            pl.BlockSpec((rows, fp), lambda k, j: (j, 0)),
            pl.BlockSpec((fp, 64), lambda k, j: (0, 0)),
            pl.BlockSpec((fp, 64), lambda k, j: (0, 0)),
            pl.BlockSpec((1, 64), lambda k, j: (0, 0)),
        ],
        out_specs=[
            pl.BlockSpec((1, 64), lambda k, j: (0, 0)),
            pl.BlockSpec((1, 64), lambda k, j: (0, 0)),
        ],
        out_shape=[
            jax.ShapeDtypeStruct((1, 64), jnp.float32),
            jax.ShapeDtypeStruct((1, 64), jnp.float32),
        ],
    )(pos_g, pos_pad, wa, wb, b1)


def _l2_body(pg_ref, p_ref, wa_ref, wb_ref, b_ref, a_ref, bb_ref, w_ref,
             b2_ref, e_ref, s_ref, q_ref):
    first = (pl.program_id(0) == 0) & (pl.program_id(1) == 0)

    @pl.when(first)
    def _():
        s_ref[...] = jnp.zeros_like(s_ref)
        q_ref[...] = jnp.zeros_like(q_ref)

    pre = _pre1(p_ref[...], pg_ref[0], wa_ref, wb_ref, b_ref)
    r = jnp.maximum(pre * a_ref[...] + bb_ref[...], 0.0)
    h = jnp.dot(r, w_ref[...], preferred_element_type=jnp.float32) + b2_ref[...]
    e_ref[...] = h[None]
    s_ref[...] += jnp.sum(h, axis=0, keepdims=True)
    q_ref[...] += jnp.sum(h * h, axis=0, keepdims=True)


def _l2(pos_g, pos_pad, wa, wb, b1, a1, b1f, w2, b2, rows=1024):
    fp = pos_pad.shape[1]
    grid = (K, N // rows)
    return pl.pallas_call(
        _l2_body,
        grid=grid,
        in_specs=[
            pl.BlockSpec((1, rows, fp), lambda k, j: (k, j, 0)),
            pl.BlockSpec((rows, fp), lambda k, j: (j, 0)),
            pl.BlockSpec((fp, 64), lambda k, j: (0, 0)),
            pl.BlockSpec((fp, 64), lambda k, j: (0, 0)),
            pl.BlockSpec((1, 64), lambda k, j: (0, 0)),
            pl.BlockSpec((1, 64), lambda k, j: (0, 0)),
            pl.BlockSpec((1, 64), lambda k, j: (0, 0)),
            pl.BlockSpec((64, 64), lambda k, j: (0, 0)),
            pl.BlockSpec((1, 64), lambda k, j: (0, 0)),
        ],
        out_specs=[
            pl.BlockSpec((1, rows, 64), lambda k, j: (k, j, 0)),
            pl.BlockSpec((1, 64), lambda k, j: (0, 0)),
            pl.BlockSpec((1, 64), lambda k, j: (0, 0)),
        ],
        out_shape=[
            jax.ShapeDtypeStruct((K, N, 64), jnp.float32),
            jax.ShapeDtypeStruct((1, 64), jnp.float32),
            jax.ShapeDtypeStruct((1, 64), jnp.float32),
        ],
    )(pos_g, pos_pad, wa, wb, b1, a1, b1f, w2, b2)


def _l3_body(e_ref, a_ref, bb_ref, w_ref, b3_ref, x1_ref):
    acc = None
    for k in range(K):
        r = jnp.maximum(e_ref[k] * a_ref[...] + bb_ref[...], 0.0)
        h = jnp.dot(r, w_ref[...], preferred_element_type=jnp.float32)
        acc = h if acc is None else jnp.maximum(acc, h)
    x1_ref[...] = acc + b3_ref[...]


def _l3(e2, a2, b2f, w3, b3, rows=256):
    f = w3.shape[0]
    grid = (N // rows,)
    return pl.pallas_call(
        _l3_body,
        grid=grid,
        in_specs=[
            pl.BlockSpec((K, rows, f), lambda i: (0, i, 0)),
            pl.BlockSpec((1, f), lambda i: (0, 0)),
            pl.BlockSpec((1, f), lambda i: (0, 0)),
            pl.BlockSpec((f, f), lambda i: (0, 0)),
            pl.BlockSpec((1, f), lambda i: (0, 0)),
        ],
        out_specs=pl.BlockSpec((rows, f), lambda i: (i, 0)),
        out_shape=jax.ShapeDtypeStruct((N, f), jnp.float32),
    )(e2, a2, b2f, w3, b3)


def _conv2_body(xg_ref, x_ref, wa_ref, wb_ref, b4_ref, x2_ref):
    x = x_ref[...]
    a = jnp.dot(x, wa_ref[...], preferred_element_type=jnp.float32) + b4_ref[...]
    acc = None
    for k in range(K):
        h = a + jnp.dot(xg_ref[k] - x, wb_ref[...], preferred_element_type=jnp.float32)
        acc = h if acc is None else jnp.maximum(acc, h)
    x2_ref[...] = acc


def _conv2(x1_g, x1, wa, wb, b4, rows=256):
    grid = (N // rows,)
    return pl.pallas_call(
        _conv2_body,
        grid=grid,
        in_specs=[
            pl.BlockSpec((K, rows, 64), lambda i: (0, i, 0)),
            pl.BlockSpec((rows, 64), lambda i: (i, 0)),
            pl.BlockSpec((64, 128), lambda i: (0, 0)),
            pl.BlockSpec((64, 128), lambda i: (0, 0)),
            pl.BlockSpec((1, 128), lambda i: (0, 0)),
        ],
        out_specs=pl.BlockSpec((rows, 128), lambda i: (i, 0)),
        out_shape=jax.ShapeDtypeStruct((N, 128), jnp.float32),
    )(x1_g, x1, wa, wb, b4)


def _pool_body(x1_ref, x2_ref, bc_ref, wa_ref, wb_ref, b5_ref, p_ref):
    @pl.when(pl.program_id(0) == 0)
    def _():
        p_ref[...] = jnp.full_like(p_ref, -jnp.inf)

    o = (jnp.dot(x1_ref[...], wa_ref[...], preferred_element_type=jnp.float32)
         + jnp.dot(x2_ref[...], wb_ref[...], preferred_element_type=jnp.float32)
         + b5_ref[...])
    b = bc_ref[...]
    for g in range(NUM_GRAPHS):
        mg = jnp.max(jnp.where(b == g, o, -jnp.inf), axis=0, keepdims=True)
        p_ref[g:g + 1, :] = jnp.maximum(p_ref[g:g + 1, :], mg)


def _pool(x1, x2, batch_col, w5a, w5b, b5, rows=256):
    grid = (N // rows,)
    return pl.pallas_call(
        _pool_body,
        grid=grid,
        in_specs=[
            pl.BlockSpec((rows, x1.shape[1]), lambda i: (i, 0)),
            pl.BlockSpec((rows, x2.shape[1]), lambda i: (i, 0)),
            pl.BlockSpec((rows, 1), lambda i: (i, 0)),
            pl.BlockSpec(w5a.shape, lambda i: (0, 0)),
            pl.BlockSpec(w5b.shape, lambda i: (0, 0)),
            pl.BlockSpec((1, b5.shape[1]), lambda i: (0, 0)),
        ],
        out_specs=pl.BlockSpec((NUM_GRAPHS, w5a.shape[1]), lambda i: (0, 0)),
        out_shape=jax.ShapeDtypeStruct((NUM_GRAPHS, w5a.shape[1]), jnp.float32),
    )(x1, x2, batch_col, w5a, w5b, b5)


def _head_body(p_ref, w6_ref, b6_ref, w7_ref, b7_ref, w8_ref, b8_ref, o_ref):
    h = jnp.maximum(jnp.dot(p_ref[...], w6_ref[...], preferred_element_type=jnp.float32) + b6_ref[...], 0.0)
    h = jnp.maximum(jnp.dot(h, w7_ref[...], preferred_element_type=jnp.float32) + b7_ref[...], 0.0)
    o_ref[...] = jnp.dot(h, w8_ref[...], preferred_element_type=jnp.float32) + b8_ref[...]


def _head(pooled, w6, b6, w7, b7, w8, b8):
    return pl.pallas_call(
        _head_body,
        out_shape=jax.ShapeDtypeStruct((NUM_GRAPHS, OUT_DIM), jnp.float32),
    )(pooled, w6, b6.reshape(1, -1), w7, b7.reshape(1, -1), w8, b8.reshape(1, -1))


def _bn_fold(s, q, gamma, beta):
    e = float(N * K)
    m = s / e
    var = jnp.maximum(q / e - m * m, 0.0)
    a = gamma.reshape(1, -1) / jnp.sqrt(var + 1e-5)
    b = beta.reshape(1, -1) - m * a
    return a, b


def kernel(pos, batch, W1, b1, g1, bt1, W2, b2, g2, bt2, W3, b3, W4, b4,
           W5, b5, W6, b6, W7, b7, W8, b8):
    batch = batch.astype(jnp.int32)
    batch_col = batch.reshape(N, 1)
    batch_row = batch.reshape(1, N)

    pos_pad = jnp.pad(pos, ((0, 0), (0, 13)))
    w1a = jnp.pad(W1[:3], ((0, 13), (0, 0)))
    w1b = jnp.pad(W1[3:], ((0, 13), (0, 0)))
    b1r = b1.reshape(1, -1)

    info = _knn_window_info(batch)
    idx1 = _knn(info, pos_pad, pos_pad.T, batch_col, batch_row)
    idx1_flat = idx1.T.reshape(-1)

    pos_g = _make_sc_gather(N * K, 16, 512)(pos_pad, idx1_flat).reshape(K, N, 16)

    s1, q1 = _stats1(pos_g, pos_pad, w1a, w1b, b1r)
    a1, b1f = _bn_fold(s1, q1, g1, bt1)

    e2, s2, q2 = _l2(pos_g, pos_pad, w1a, w1b, b1r, a1, b1f, W2, b2.reshape(1, -1))
    a2, b2f = _bn_fold(s2, q2, g2, bt2)

    x1 = _l3(e2, a2, b2f, W3, b3.reshape(1, -1))

    idx2 = _knn(info, x1, x1.T, batch_col, batch_row)
    idx2_flat = idx2.T.reshape(-1)

    x1_g = _make_sc_gather(N * K, 64, 512)(x1, idx2_flat).reshape(K, N, 64)

    x2 = _conv2(x1_g, x1, W4[:64], W4[64:], b4.reshape(1, -1))

    pooled = _pool(x1, x2, batch_col, W5[:64], W5[64:], b5.reshape(1, -1))
    return _head(pooled, W6, b6, W7, b7, W8, b8)


# speedup vs baseline: 1.8958x; 1.8958x over previous
import functools

import jax
import jax.numpy as jnp
from jax.experimental import pallas as pl
from jax.experimental.pallas import tpu as pltpu
from jax.experimental.pallas import tpu_sc as plsc

N = 8192
K = 20
NUM_GRAPHS = 8
OUT_DIM = 40

_BIG = 1e30
_BIG2 = 2e30


_WINS = (1536, 2560, N)


def _extract_topk(d, lane, idx_ref):
    cols = []
    for _ in range(K):
        m = jnp.min(d, axis=1, keepdims=True)
        am = jnp.min(jnp.where(d <= m, lane, N), axis=1, keepdims=True)
        cols.append(am)
        d = jnp.where(lane == am, _BIG2, d)
    idx_ref[...] = jnp.concatenate(cols, axis=1)


def _knn_body(info_ref, x_ref, xt_ref, bc_ref, br_ref, idx_ref, *, rows):
    pid = pl.program_id(0)
    start = pl.multiple_of(info_ref[0, pid], 256)
    sel = info_ref[1, pid]
    x = x_ref[...]
    sqc = jnp.sum(x * x, axis=1, keepdims=True)
    bc = bc_ref[...]

    def _tier(win, s):
        xt = xt_ref[:, pl.ds(s, win)] if win < N else xt_ref[...]
        br = br_ref[:, pl.ds(s, win)] if win < N else br_ref[...]
        sqr = jnp.sum(xt * xt, axis=0, keepdims=True)
        d = sqc + sqr - 2.0 * jnp.dot(x, xt, preferred_element_type=jnp.float32)
        d = jnp.where(bc != br, _BIG, d)
        lane = jax.lax.broadcasted_iota(jnp.int32, (rows, win), 1) + s
        _extract_topk(d, lane, idx_ref)

    for t, w in enumerate(_WINS):
        s = start if w < N else 0

        @pl.when(sel == t)
        def _(w=w, s=s):
            _tier(w, s)


def _knn(info, x, xt, batch_col, batch_row, rows=128):
    f = x.shape[1]
    grid = N // rows
    return pl.pallas_call(
        functools.partial(_knn_body, rows=rows),
        grid_spec=pltpu.PrefetchScalarGridSpec(
            num_scalar_prefetch=1,
            grid=(grid,),
            in_specs=[
                pl.BlockSpec((rows, f), lambda i, inf: (i, 0)),
                pl.BlockSpec((f, N), lambda i, inf: (0, 0)),
                pl.BlockSpec((rows, 1), lambda i, inf: (i, 0)),
                pl.BlockSpec((1, N), lambda i, inf: (0, 0)),
            ],
            out_specs=pl.BlockSpec((rows, K), lambda i, inf: (i, 0)),
        ),
        out_shape=jax.ShapeDtypeStruct((N, K), jnp.int32),
    )(info, x, xt, batch_col, batch_row)


def _knn_window_info(batch, rows=128):
    br = batch.reshape(N // rows, rows)
    seg_start = jnp.searchsorted(batch, jnp.arange(NUM_GRAPHS), side="left")
    seg_end = jnp.searchsorted(batch, jnp.arange(NUM_GRAPHS) + 1, side="left")
    cs = seg_start[br[:, 0]]
    ce = seg_end[br[:, -1]]
    s = (cs // 256) * 256
    sel = jnp.full(s.shape, len(_WINS) - 1, jnp.int32)
    for t in range(len(_WINS) - 2, -1, -1):
        st = jnp.minimum(s, N - _WINS[t])
        sel = jnp.where(ce - st <= _WINS[t], t, sel)
    s_final = jnp.minimum(s, N - jnp.asarray(_WINS)[sel])
    return jnp.stack([s_final.astype(jnp.int32), sel])


def _make_sc_gather(b_total, d, chunk):
    info = plsc.get_sparse_core_info()
    nc, ns = info.num_cores, info.num_subcores
    nw = nc * ns
    b_per_w = b_total // nw
    nchunks = b_per_w // chunk
    assert b_per_w % chunk == 0 and chunk % 8 == 0
    mesh = plsc.VectorSubcoreMesh(core_axis_name="c", subcore_axis_name="s")

    @functools.partial(
        pl.kernel,
        mesh=mesh,
        compiler_params=pltpu.CompilerParams(use_tc_tiling_on_sc=False),
        out_type=jax.ShapeDtypeStruct((b_total, d), jnp.float32),
        scratch_types=[
            pltpu.VMEM((chunk,), jnp.int32),
            pltpu.VMEM((chunk, d), jnp.float32),
            pltpu.SemaphoreType.DMA,
        ],
    )
    def gather(table_hbm, idx_hbm, out_hbm, idx_v, rows_v, sem):
        wid = jax.lax.axis_index("s") * nc + jax.lax.axis_index("c")
        base = wid * b_per_w
        for c in range(nchunks):
            off = base + c * chunk
            pltpu.sync_copy(idx_hbm.at[pl.ds(off, chunk)], idx_v)
            pltpu.async_copy(table_hbm.at[idx_v], rows_v, sem).wait()
            pltpu.sync_copy(rows_v, out_hbm.at[pl.ds(off, chunk)])

    return gather


def _pre1(pos_blk, posg_blk, wa_ref, wb_ref, b_ref):
    return (jnp.dot(pos_blk, wa_ref[...], preferred_element_type=jnp.float32)
            + jnp.dot(posg_blk - pos_blk, wb_ref[...], preferred_element_type=jnp.float32)
            + b_ref[...])


def _stats1_body(pg_ref, p_ref, wa_ref, wb_ref, b_ref, s_ref, q_ref):
    first = (pl.program_id(0) == 0) & (pl.program_id(1) == 0)

    @pl.when(first)
    def _():
        s_ref[...] = jnp.zeros_like(s_ref)
        q_ref[...] = jnp.zeros_like(q_ref)

    pre = _pre1(p_ref[...], pg_ref[0], wa_ref, wb_ref, b_ref)
    s_ref[...] += jnp.sum(pre, axis=0, keepdims=True)
    q_ref[...] += jnp.sum(pre * pre, axis=0, keepdims=True)


def _stats1(pos_g, pos_pad, wa, wb, b1, rows=1024):
    fp = pos_pad.shape[1]
    grid = (K, N // rows)
    return pl.pallas_call(
        _stats1_body,
        grid=grid,
        in_specs=[
            pl.BlockSpec((1, rows, fp), lambda k, j: (k, j, 0)),
            pl.BlockSpec((rows, fp), lambda k, j: (j, 0)),
            pl.BlockSpec((fp, 64), lambda k, j: (0, 0)),
            pl.BlockSpec((fp, 64), lambda k, j: (0, 0)),
            pl.BlockSpec((1, 64), lambda k, j: (0, 0)),
        ],
        out_specs=[
            pl.BlockSpec((1, 64), lambda k, j: (0, 0)),
            pl.BlockSpec((1, 64), lambda k, j: (0, 0)),
        ],
        out_shape=[
            jax.ShapeDtypeStruct((1, 64), jnp.float32),
            jax.ShapeDtypeStruct((1, 64), jnp.float32),
        ],
    )(pos_g, pos_pad, wa, wb, b1)


def _l2_body(pg_ref, p_ref, wa_ref, wb_ref, b_ref, a_ref, bb_ref, w_ref,
             b2_ref, e_ref, s_ref, q_ref):
    first = (pl.program_id(0) == 0) & (pl.program_id(1) == 0)

    @pl.when(first)
    def _():
        s_ref[...] = jnp.zeros_like(s_ref)
        q_ref[...] = jnp.zeros_like(q_ref)

    pre = _pre1(p_ref[...], pg_ref[0], wa_ref, wb_ref, b_ref)
    r = jnp.maximum(pre * a_ref[...] + bb_ref[...], 0.0)
    h = jnp.dot(r, w_ref[...], preferred_element_type=jnp.float32) + b2_ref[...]
    e_ref[...] = h[None]
    s_ref[...] += jnp.sum(h, axis=0, keepdims=True)
    q_ref[...] += jnp.sum(h * h, axis=0, keepdims=True)


def _l2(pos_g, pos_pad, wa, wb, b1, a1, b1f, w2, b2, rows=1024):
    fp = pos_pad.shape[1]
    grid = (K, N // rows)
    return pl.pallas_call(
        _l2_body,
        grid=grid,
        in_specs=[
            pl.BlockSpec((1, rows, fp), lambda k, j: (k, j, 0)),
            pl.BlockSpec((rows, fp), lambda k, j: (j, 0)),
            pl.BlockSpec((fp, 64), lambda k, j: (0, 0)),
            pl.BlockSpec((fp, 64), lambda k, j: (0, 0)),
            pl.BlockSpec((1, 64), lambda k, j: (0, 0)),
            pl.BlockSpec((1, 64), lambda k, j: (0, 0)),
            pl.BlockSpec((1, 64), lambda k, j: (0, 0)),
            pl.BlockSpec((64, 64), lambda k, j: (0, 0)),
            pl.BlockSpec((1, 64), lambda k, j: (0, 0)),
        ],
        out_specs=[
            pl.BlockSpec((1, rows, 64), lambda k, j: (k, j, 0)),
            pl.BlockSpec((1, 64), lambda k, j: (0, 0)),
            pl.BlockSpec((1, 64), lambda k, j: (0, 0)),
        ],
        out_shape=[
            jax.ShapeDtypeStruct((K, N, 64), jnp.float32),
            jax.ShapeDtypeStruct((1, 64), jnp.float32),
            jax.ShapeDtypeStruct((1, 64), jnp.float32),
        ],
    )(pos_g, pos_pad, wa, wb, b1, a1, b1f, w2, b2)


def _l3_body(e_ref, a_ref, bb_ref, w_ref, b3_ref, x1_ref):
    acc = None
    for k in range(K):
        r = jnp.maximum(e_ref[k] * a_ref[...] + bb_ref[...], 0.0)
        h = jnp.dot(r, w_ref[...], preferred_element_type=jnp.float32)
        acc = h if acc is None else jnp.maximum(acc, h)
    x1_ref[...] = acc + b3_ref[...]


def _l3(e2, a2, b2f, w3, b3, rows=256):
    f = w3.shape[0]
    grid = (N // rows,)
    return pl.pallas_call(
        _l3_body,
        grid=grid,
        in_specs=[
            pl.BlockSpec((K, rows, f), lambda i: (0, i, 0)),
            pl.BlockSpec((1, f), lambda i: (0, 0)),
            pl.BlockSpec((1, f), lambda i: (0, 0)),
            pl.BlockSpec((f, f), lambda i: (0, 0)),
            pl.BlockSpec((1, f), lambda i: (0, 0)),
        ],
        out_specs=pl.BlockSpec((rows, f), lambda i: (i, 0)),
        out_shape=jax.ShapeDtypeStruct((N, f), jnp.float32),
    )(e2, a2, b2f, w3, b3)


def _conv2_body(xg_ref, x_ref, wa_ref, wb_ref, b4_ref, x2_ref):
    x = x_ref[...]
    a = jnp.dot(x, wa_ref[...], preferred_element_type=jnp.float32) + b4_ref[...]
    acc = None
    for k in range(K):
        h = a + jnp.dot(xg_ref[k] - x, wb_ref[...], preferred_element_type=jnp.float32)
        acc = h if acc is None else jnp.maximum(acc, h)
    x2_ref[...] = acc


def _conv2(x1_g, x1, wa, wb, b4, rows=256):
    grid = (N // rows,)
    return pl.pallas_call(
        _conv2_body,
        grid=grid,
        in_specs=[
            pl.BlockSpec((K, rows, 64), lambda i: (0, i, 0)),
            pl.BlockSpec((rows, 64), lambda i: (i, 0)),
            pl.BlockSpec((64, 128), lambda i: (0, 0)),
            pl.BlockSpec((64, 128), lambda i: (0, 0)),
            pl.BlockSpec((1, 128), lambda i: (0, 0)),
        ],
        out_specs=pl.BlockSpec((rows, 128), lambda i: (i, 0)),
        out_shape=jax.ShapeDtypeStruct((N, 128), jnp.float32),
    )(x1_g, x1, wa, wb, b4)


def _pool_body(x1_ref, x2_ref, bc_ref, wa_ref, wb_ref, b5_ref, p_ref):
    @pl.when(pl.program_id(0) == 0)
    def _():
        p_ref[...] = jnp.full_like(p_ref, -jnp.inf)

    o = (jnp.dot(x1_ref[...], wa_ref[...], preferred_element_type=jnp.float32)
         + jnp.dot(x2_ref[...], wb_ref[...], preferred_element_type=jnp.float32)
         + b5_ref[...])
    b = bc_ref[...]
    for g in range(NUM_GRAPHS):
        mg = jnp.max(jnp.where(b == g, o, -jnp.inf), axis=0, keepdims=True)
        p_ref[g:g + 1, :] = jnp.maximum(p_ref[g:g + 1, :], mg)


def _pool(x1, x2, batch_col, w5a, w5b, b5, rows=256):
    grid = (N // rows,)
    return pl.pallas_call(
        _pool_body,
        grid=grid,
        in_specs=[
            pl.BlockSpec((rows, x1.shape[1]), lambda i: (i, 0)),
            pl.BlockSpec((rows, x2.shape[1]), lambda i: (i, 0)),
            pl.BlockSpec((rows, 1), lambda i: (i, 0)),
            pl.BlockSpec(w5a.shape, lambda i: (0, 0)),
            pl.BlockSpec(w5b.shape, lambda i: (0, 0)),
            pl.BlockSpec((1, b5.shape[1]), lambda i: (0, 0)),
        ],
        out_specs=pl.BlockSpec((NUM_GRAPHS, w5a.shape[1]), lambda i: (0, 0)),
        out_shape=jax.ShapeDtypeStruct((NUM_GRAPHS, w5a.shape[1]), jnp.float32),
    )(x1, x2, batch_col, w5a, w5b, b5)


def _head_body(p_ref, w6_ref, b6_ref, w7_ref, b7_ref, w8_ref, b8_ref, o_ref):
    h = jnp.maximum(jnp.dot(p_ref[...], w6_ref[...], preferred_element_type=jnp.float32) + b6_ref[...], 0.0)
    h = jnp.maximum(jnp.dot(h, w7_ref[...], preferred_element_type=jnp.float32) + b7_ref[...], 0.0)
    o_ref[...] = jnp.dot(h, w8_ref[...], preferred_element_type=jnp.float32) + b8_ref[...]


def _head(pooled, w6, b6, w7, b7, w8, b8):
    return pl.pallas_call(
        _head_body,
        out_shape=jax.ShapeDtypeStruct((NUM_GRAPHS, OUT_DIM), jnp.float32),
    )(pooled, w6, b6.reshape(1, -1), w7, b7.reshape(1, -1), w8, b8.reshape(1, -1))


def _bn_fold(s, q, gamma, beta):
    e = float(N * K)
    m = s / e
    var = jnp.maximum(q / e - m * m, 0.0)
    a = gamma.reshape(1, -1) / jnp.sqrt(var + 1e-5)
    b = beta.reshape(1, -1) - m * a
    return a, b


def kernel(pos, batch, W1, b1, g1, bt1, W2, b2, g2, bt2, W3, b3, W4, b4,
           W5, b5, W6, b6, W7, b7, W8, b8):
    batch = batch.astype(jnp.int32)
    batch_col = batch.reshape(N, 1)
    batch_row = batch.reshape(1, N)

    pos_pad = jnp.pad(pos, ((0, 0), (0, 13)))
    w1a = jnp.pad(W1[:3], ((0, 13), (0, 0)))
    w1b = jnp.pad(W1[3:], ((0, 13), (0, 0)))
    b1r = b1.reshape(1, -1)

    info = _knn_window_info(batch)
    idx1 = _knn(info, pos_pad, pos_pad.T, batch_col, batch_row)
    idx1_flat = idx1.T.reshape(-1)

    pos_g = _make_sc_gather(N * K, 16, 512)(pos_pad, idx1_flat).reshape(K, N, 16)

    s1, q1 = _stats1(pos_g, pos_pad, w1a, w1b, b1r)
    a1, b1f = _bn_fold(s1, q1, g1, bt1)

    e2, s2, q2 = _l2(pos_g, pos_pad, w1a, w1b, b1r, a1, b1f, W2, b2.reshape(1, -1))
    a2, b2f = _bn_fold(s2, q2, g2, bt2)

    x1 = _l3(e2, a2, b2f, W3, b3.reshape(1, -1))

    idx2 = _knn(info, x1, x1.T, batch_col, batch_row)
    idx2_flat = idx2.T.reshape(-1)

    x1_g = _make_sc_gather(N * K, 64, 512)(x1, idx2_flat).reshape(K, N, 64)

    x2 = _conv2(x1_g, x1, W4[:64], W4[64:], b4.reshape(1, -1))

    pooled = _pool(x1, x2, batch_col, W5[:64], W5[64:], b5.reshape(1, -1))
    return _head(pooled, W6, b6, W7, b7, W8, b8)


# speedup vs baseline: 1.8999x; 1.0022x over previous
import functools

import jax
import jax.numpy as jnp
from jax.experimental import pallas as pl
from jax.experimental.pallas import tpu as pltpu
from jax.experimental.pallas import tpu_sc as plsc

N = 8192
K = 20
NUM_GRAPHS = 8
OUT_DIM = 40

_BIG = 1e30
_BIG2 = 2e30


_WINS = (1536, 2560, N)


def _extract_topk(d, lane, idx_ref):
    cols = []
    for _ in range(K):
        m = jnp.min(d, axis=1, keepdims=True)
        am = jnp.min(jnp.where(d <= m, lane, N), axis=1, keepdims=True)
        cols.append(am)
        d = jnp.where(lane == am, _BIG2, d)
    idx_ref[...] = jnp.concatenate(cols, axis=1)


def _knn_body(info_ref, x_ref, xt_ref, bc_ref, br_ref, idx_ref, *, rows):
    pid = pl.program_id(0)
    start = pl.multiple_of(info_ref[0, pid], 256)
    sel = info_ref[1, pid]
    x = x_ref[...]
    sqc = jnp.sum(x * x, axis=1, keepdims=True)
    bc = bc_ref[...]

    def _tier(win, s):
        xt = xt_ref[:, pl.ds(s, win)] if win < N else xt_ref[...]
        br = br_ref[:, pl.ds(s, win)] if win < N else br_ref[...]
        sqr = jnp.sum(xt * xt, axis=0, keepdims=True)
        d = sqc + sqr - 2.0 * jnp.dot(x, xt, preferred_element_type=jnp.float32)
        d = jnp.where(bc != br, _BIG, d)
        lane = jax.lax.broadcasted_iota(jnp.int32, (rows, win), 1) + s
        _extract_topk(d, lane, idx_ref)

    for t, w in enumerate(_WINS):
        s = start if w < N else 0

        @pl.when(sel == t)
        def _(w=w, s=s):
            _tier(w, s)


def _knn(info, x, xt, batch_col, batch_row, rows=128):
    f = x.shape[1]
    grid = N // rows
    return pl.pallas_call(
        functools.partial(_knn_body, rows=rows),
        grid_spec=pltpu.PrefetchScalarGridSpec(
            num_scalar_prefetch=1,
            grid=(grid,),
            in_specs=[
                pl.BlockSpec((rows, f), lambda i, inf: (i, 0)),
                pl.BlockSpec((f, N), lambda i, inf: (0, 0)),
                pl.BlockSpec((rows, 1), lambda i, inf: (i, 0)),
                pl.BlockSpec((1, N), lambda i, inf: (0, 0)),
            ],
            out_specs=pl.BlockSpec((rows, K), lambda i, inf: (i, 0)),
        ),
        out_shape=jax.ShapeDtypeStruct((N, K), jnp.int32),
    )(info, x, xt, batch_col, batch_row)


def _knn_window_info(batch, rows=128):
    br = batch.reshape(N // rows, rows)
    seg_start = jnp.searchsorted(batch, jnp.arange(NUM_GRAPHS), side="left")
    seg_end = jnp.searchsorted(batch, jnp.arange(NUM_GRAPHS) + 1, side="left")
    cs = seg_start[br[:, 0]]
    ce = seg_end[br[:, -1]]
    s = (cs // 256) * 256
    sel = jnp.full(s.shape, len(_WINS) - 1, jnp.int32)
    for t in range(len(_WINS) - 2, -1, -1):
        st = jnp.minimum(s, N - _WINS[t])
        sel = jnp.where(ce - st <= _WINS[t], t, sel)
    s_final = jnp.minimum(s, N - jnp.asarray(_WINS)[sel])
    return jnp.stack([s_final.astype(jnp.int32), sel])


def _make_sc_gather(b_total, d, chunk):
    info = plsc.get_sparse_core_info()
    nc, ns = info.num_cores, info.num_subcores
    nw = nc * ns
    b_per_w = b_total // nw
    nchunks = b_per_w // chunk
    assert b_per_w % chunk == 0 and chunk % 8 == 0
    mesh = plsc.VectorSubcoreMesh(core_axis_name="c", subcore_axis_name="s")

    @functools.partial(
        pl.kernel,
        mesh=mesh,
        compiler_params=pltpu.CompilerParams(use_tc_tiling_on_sc=False),
        out_type=jax.ShapeDtypeStruct((b_total, d), jnp.float32),
        scratch_types=[
            pltpu.VMEM((chunk,), jnp.int32),
            pltpu.VMEM((chunk, d), jnp.float32),
            pltpu.SemaphoreType.DMA,
        ],
    )
    def gather(table_hbm, idx_hbm, out_hbm, idx_v, rows_v, sem):
        wid = jax.lax.axis_index("s") * nc + jax.lax.axis_index("c")
        base = wid * b_per_w
        for c in range(nchunks):
            off = base + c * chunk
            pltpu.sync_copy(idx_hbm.at[pl.ds(off, chunk)], idx_v)
            pltpu.async_copy(table_hbm.at[idx_v], rows_v, sem).wait()
            pltpu.sync_copy(rows_v, out_hbm.at[pl.ds(off, chunk)])

    return gather


def _pre1(pos_blk, posg_blk, wa_ref, wb_ref, b_ref):
    return (jnp.dot(pos_blk, wa_ref[...], preferred_element_type=jnp.float32)
            + jnp.dot(posg_blk - pos_blk, wb_ref[...], preferred_element_type=jnp.float32)
            + b_ref[...])


def _stats1_body(pg_ref, p_ref, wa_ref, wb_ref, b_ref, s_ref, q_ref):
    first = (pl.program_id(0) == 0) & (pl.program_id(1) == 0)

    @pl.when(first)
    def _():
        s_ref[...] = jnp.zeros_like(s_ref)
        q_ref[...] = jnp.zeros_like(q_ref)

    pre = _pre1(p_ref[...], pg_ref[0], wa_ref, wb_ref, b_ref)
    s_ref[...] += jnp.sum(pre, axis=0, keepdims=True)
    q_ref[...] += jnp.sum(pre * pre, axis=0, keepdims=True)


def _stats1(pos_g, pos_pad, wa, wb, b1, rows=1024):
    fp = pos_pad.shape[1]
    grid = (K, N // rows)
    return pl.pallas_call(
        _stats1_body,
        grid=grid,
        in_specs=[
            pl.BlockSpec((1, rows, fp), lambda k, j: (k, j, 0)),
            pl.BlockSpec((rows, fp), lambda k, j: (j, 0)),
            pl.BlockSpec((fp, 64), lambda k, j: (0, 0)),
            pl.BlockSpec((fp, 64), lambda k, j: (0, 0)),
            pl.BlockSpec((1, 64), lambda k, j: (0, 0)),
        ],
        out_specs=[
            pl.BlockSpec((1, 64), lambda k, j: (0, 0)),
            pl.BlockSpec((1, 64), lambda k, j: (0, 0)),
        ],
        out_shape=[
            jax.ShapeDtypeStruct((1, 64), jnp.float32),
            jax.ShapeDtypeStruct((1, 64), jnp.float32),
        ],
    )(pos_g, pos_pad, wa, wb, b1)


def _bn_fold_k(s, q, g, bt):
    e = float(N * K)
    m = s / e
    var = jnp.maximum(q / e - m * m, 0.0)
    a = g / jnp.sqrt(var + 1e-5)
    return a, bt - m * a


def _l2_body(pg_ref, p_ref, wa_ref, wb_ref, b_ref, s1_ref, q1_ref, g_ref,
             bt_ref, w_ref, b2_ref, e_ref, s_ref, q_ref):
    first = (pl.program_id(0) == 0) & (pl.program_id(1) == 0)

    @pl.when(first)
    def _():
        s_ref[...] = jnp.zeros_like(s_ref)
        q_ref[...] = jnp.zeros_like(q_ref)

    a1, b1f = _bn_fold_k(s1_ref[...], q1_ref[...], g_ref[...], bt_ref[...])
    pre = _pre1(p_ref[...], pg_ref[0], wa_ref, wb_ref, b_ref)
    r = jnp.maximum(pre * a1 + b1f, 0.0)
    h = jnp.dot(r, w_ref[...], preferred_element_type=jnp.float32) + b2_ref[...]
    e_ref[...] = h[None]
    s_ref[...] += jnp.sum(h, axis=0, keepdims=True)
    q_ref[...] += jnp.sum(h * h, axis=0, keepdims=True)


def _l2(pos_g, pos_pad, wa, wb, b1, s1, q1, g1, bt1, w2, b2, rows=1024):
    fp = pos_pad.shape[1]
    grid = (K, N // rows)
    return pl.pallas_call(
        _l2_body,
        grid=grid,
        in_specs=[
            pl.BlockSpec((1, rows, fp), lambda k, j: (k, j, 0)),
            pl.BlockSpec((rows, fp), lambda k, j: (j, 0)),
            pl.BlockSpec((fp, 64), lambda k, j: (0, 0)),
            pl.BlockSpec((fp, 64), lambda k, j: (0, 0)),
            pl.BlockSpec((1, 64), lambda k, j: (0, 0)),
            pl.BlockSpec((1, 64), lambda k, j: (0, 0)),
            pl.BlockSpec((1, 64), lambda k, j: (0, 0)),
            pl.BlockSpec((1, 64), lambda k, j: (0, 0)),
            pl.BlockSpec((1, 64), lambda k, j: (0, 0)),
            pl.BlockSpec((64, 64), lambda k, j: (0, 0)),
            pl.BlockSpec((1, 64), lambda k, j: (0, 0)),
        ],
        out_specs=[
            pl.BlockSpec((1, rows, 64), lambda k, j: (k, j, 0)),
            pl.BlockSpec((1, 64), lambda k, j: (0, 0)),
            pl.BlockSpec((1, 64), lambda k, j: (0, 0)),
        ],
        out_shape=[
            jax.ShapeDtypeStruct((K, N, 64), jnp.float32),
            jax.ShapeDtypeStruct((1, 64), jnp.float32),
            jax.ShapeDtypeStruct((1, 64), jnp.float32),
        ],
    )(pos_g, pos_pad, wa, wb, b1, s1, q1, g1, bt1, w2, b2)


def _l3_body(e_ref, s2_ref, q2_ref, g_ref, bt_ref, w_ref, b3_ref,
             x1_ref, x1t_ref):
    a2, b2f = _bn_fold_k(s2_ref[...], q2_ref[...], g_ref[...], bt_ref[...])
    acc = None
    for k in range(K):
        r = jnp.maximum(e_ref[k] * a2 + b2f, 0.0)
        h = jnp.dot(r, w_ref[...], preferred_element_type=jnp.float32)
        acc = h if acc is None else jnp.maximum(acc, h)
    x1 = acc + b3_ref[...]
    x1_ref[...] = x1
    x1t_ref[...] = x1.T


def _l3(e2, s2, q2, g2, bt2, w3, b3, rows=256):
    f = w3.shape[0]
    grid = (N // rows,)
    return pl.pallas_call(
        _l3_body,
        grid=grid,
        in_specs=[
            pl.BlockSpec((K, rows, f), lambda i: (0, i, 0)),
            pl.BlockSpec((1, f), lambda i: (0, 0)),
            pl.BlockSpec((1, f), lambda i: (0, 0)),
            pl.BlockSpec((1, f), lambda i: (0, 0)),
            pl.BlockSpec((1, f), lambda i: (0, 0)),
            pl.BlockSpec((f, f), lambda i: (0, 0)),
            pl.BlockSpec((1, f), lambda i: (0, 0)),
        ],
        out_specs=[
            pl.BlockSpec((rows, f), lambda i: (i, 0)),
            pl.BlockSpec((f, rows), lambda i: (0, i)),
        ],
        out_shape=[
            jax.ShapeDtypeStruct((N, f), jnp.float32),
            jax.ShapeDtypeStruct((f, N), jnp.float32),
        ],
    )(e2, s2, q2, g2, bt2, w3, b3)


def _conv2_body(xg_ref, x_ref, wa_ref, wb_ref, b4_ref, x2_ref):
    x = x_ref[...]
    a = jnp.dot(x, wa_ref[...], preferred_element_type=jnp.float32) + b4_ref[...]
    acc = None
    for k in range(K):
        h = a + jnp.dot(xg_ref[k] - x, wb_ref[...], preferred_element_type=jnp.float32)
        acc = h if acc is None else jnp.maximum(acc, h)
    x2_ref[...] = acc


def _conv2(x1_g, x1, wa, wb, b4, rows=256):
    grid = (N // rows,)
    return pl.pallas_call(
        _conv2_body,
        grid=grid,
        in_specs=[
            pl.BlockSpec((K, rows, 64), lambda i: (0, i, 0)),
            pl.BlockSpec((rows, 64), lambda i: (i, 0)),
            pl.BlockSpec((64, 128), lambda i: (0, 0)),
            pl.BlockSpec((64, 128), lambda i: (0, 0)),
            pl.BlockSpec((1, 128), lambda i: (0, 0)),
        ],
        out_specs=pl.BlockSpec((rows, 128), lambda i: (i, 0)),
        out_shape=jax.ShapeDtypeStruct((N, 128), jnp.float32),
    )(x1_g, x1, wa, wb, b4)


def _pool_head_body(x1_ref, x2_ref, bc_ref, wa_ref, wb_ref, b5_ref, w6_ref,
                    b6_ref, w7_ref, b7_ref, w8_ref, b8_ref, o_ref, p_ref):
    @pl.when(pl.program_id(0) == 0)
    def _():
        p_ref[...] = jnp.full_like(p_ref, -jnp.inf)

    o = (jnp.dot(x1_ref[...], wa_ref[...], preferred_element_type=jnp.float32)
         + jnp.dot(x2_ref[...], wb_ref[...], preferred_element_type=jnp.float32)
         + b5_ref[...])
    b = bc_ref[...]
    for g in range(NUM_GRAPHS):
        mg = jnp.max(jnp.where(b == g, o, -jnp.inf), axis=0, keepdims=True)
        p_ref[g:g + 1, :] = jnp.maximum(p_ref[g:g + 1, :], mg)

    @pl.when(pl.program_id(0) == pl.num_programs(0) - 1)
    def _():
        h = jnp.maximum(jnp.dot(p_ref[...], w6_ref[...], preferred_element_type=jnp.float32) + b6_ref[...], 0.0)
        h = jnp.maximum(jnp.dot(h, w7_ref[...], preferred_element_type=jnp.float32) + b7_ref[...], 0.0)
        o_ref[...] = jnp.dot(h, w8_ref[...], preferred_element_type=jnp.float32) + b8_ref[...]


def _pool_head(x1, x2, batch_col, w5a, w5b, b5, w6, b6, w7, b7, w8, b8,
               rows=256):
    grid = (N // rows,)
    const = lambda i: (0, 0)
    return pl.pallas_call(
        _pool_head_body,
        grid=grid,
        in_specs=[
            pl.BlockSpec((rows, x1.shape[1]), lambda i: (i, 0)),
            pl.BlockSpec((rows, x2.shape[1]), lambda i: (i, 0)),
            pl.BlockSpec((rows, 1), lambda i: (i, 0)),
            pl.BlockSpec(w5a.shape, const),
            pl.BlockSpec(w5b.shape, const),
            pl.BlockSpec((1, b5.shape[1]), const),
            pl.BlockSpec(w6.shape, const),
            pl.BlockSpec((1, b6.shape[1]), const),
            pl.BlockSpec(w7.shape, const),
            pl.BlockSpec((1, b7.shape[1]), const),
            pl.BlockSpec(w8.shape, const),
            pl.BlockSpec((1, b8.shape[1]), const),
        ],
        out_specs=pl.BlockSpec((NUM_GRAPHS, OUT_DIM), const),
        out_shape=jax.ShapeDtypeStruct((NUM_GRAPHS, OUT_DIM), jnp.float32),
        scratch_shapes=[pltpu.VMEM((NUM_GRAPHS, w5a.shape[1]), jnp.float32)],
    )(x1, x2, batch_col, w5a, w5b, b5, w6, b6, w7, b7, w8, b8)


def kernel(pos, batch, W1, b1, g1, bt1, W2, b2, g2, bt2, W3, b3, W4, b4,
           W5, b5, W6, b6, W7, b7, W8, b8):
    batch = batch.astype(jnp.int32)
    batch_col = batch.reshape(N, 1)
    batch_row = batch.reshape(1, N)

    pos_pad = jnp.pad(pos, ((0, 0), (0, 13)))
    w1a = jnp.pad(W1[:3], ((0, 13), (0, 0)))
    w1b = jnp.pad(W1[3:], ((0, 13), (0, 0)))
    b1r = b1.reshape(1, -1)

    info = _knn_window_info(batch)
    idx1 = _knn(info, pos_pad, pos_pad.T, batch_col, batch_row)
    idx1_flat = idx1.T.reshape(-1)

    pos_g = _make_sc_gather(N * K, 16, 512)(pos_pad, idx1_flat).reshape(K, N, 16)

    s1, q1 = _stats1(pos_g, pos_pad, w1a, w1b, b1r)

    e2, s2, q2 = _l2(pos_g, pos_pad, w1a, w1b, b1r, s1, q1,
                     g1.reshape(1, -1), bt1.reshape(1, -1), W2,
                     b2.reshape(1, -1))

    x1, x1t = _l3(e2, s2, q2, g2.reshape(1, -1), bt2.reshape(1, -1), W3,
                  b3.reshape(1, -1))

    idx2 = _knn(info, x1, x1t, batch_col, batch_row)
    idx2_flat = idx2.T.reshape(-1)

    x1_g = _make_sc_gather(N * K, 64, 512)(x1, idx2_flat).reshape(K, N, 64)

    x2 = _conv2(x1_g, x1, W4[:64], W4[64:], b4.reshape(1, -1))

    return _pool_head(x1, x2, batch_col, W5[:64], W5[64:], b5.reshape(1, -1),
                      W6, b6.reshape(1, -1), W7, b7.reshape(1, -1), W8,
                      b8.reshape(1, -1))
